# baseline (speedup 1.0000x reference)
import functools
import math

import numpy as np

import jax
import jax.numpy as jnp
from jax.experimental import pallas as pl
from jax.experimental.pallas import tpu as pltpu


def _fused_kernel(x_hbm, w1m_h, w1a_h, qb1_h, qw2_h, qb2_h,
                  kw1_h, kb1_h, kw2_h, kb2_h, o_hbm,
                  xs, os_, w1m, w1a, qb1, qw2, qb2, kw1, kb1, kw2, kb2,
                  sin, sout, swt, *, shifts, hw, nb, height, width):
    b = pl.program_id(0)
    slot = jax.lax.rem(b, 2)

    def in_copies(i, sl):
        return [pltpu.make_async_copy(x_hbm.at[i], xs.at[sl], sin.at[sl])]

    def out_copies(i, sl):
        return [pltpu.make_async_copy(os_.at[sl], o_hbm.at[i], sout.at[sl])]

    wt_pairs = [(w1m_h, w1m), (w1a_h, w1a), (qb1_h, qb1), (qw2_h, qw2),
                (qb2_h, qb2), (kw1_h, kw1), (kb1_h, kb1), (kw2_h, kw2),
                (kb2_h, kb2)]

    @pl.when(b == 0)
    def _():
        for src, dst in wt_pairs:
            pltpu.make_async_copy(src, dst, swt).start()
        for cp in in_copies(0, 0):
            cp.start()
        for src, dst in wt_pairs:
            pltpu.make_async_copy(src, dst, swt).wait()

    @pl.when(b + 1 < nb)
    def _():
        for cp in in_copies(b + 1, 1 - slot):
            cp.start()

    for cp in in_copies(b, slot):
        cp.wait()

    x2 = xs[slot]

    v = x2
    for sh in shifts:
        v = jnp.maximum(v, pltpu.roll(v, hw - sh, 1))

    q1 = jnp.maximum(
        jnp.dot(v.astype(jnp.bfloat16), w1m[...],
                preferred_element_type=jnp.float32)
        + jnp.dot(x2.astype(jnp.bfloat16), w1a[...],
                  preferred_element_type=jnp.float32)
        + qb1[...], 0.0)

    q2 = jnp.dot(q1, qw2[...],
                 preferred_element_type=jnp.float32) + qb2[...]

    k1 = jnp.maximum(
        jnp.dot(kw1[...], q2, preferred_element_type=jnp.float32)
        + kb1[...], 0.0)
    k2 = jax.nn.sigmoid(
        jnp.dot(kw2[...], k1, preferred_element_type=jnp.float32)
        + kb2[...])

    @pl.when(b >= 2)
    def _():
        for cp in out_copies(b - 2, slot):
            cp.wait()

    os_[slot] = x2 * (k2 + 1.0)
    for cp in out_copies(b, slot):
        cp.start()

    @pl.when(b == nb - 1)
    def _():
        for cp in out_copies(b - 1, 1 - slot):
            cp.wait()
        for cp in out_copies(b, slot):
            cp.wait()


def kernel(x, qW1, qb1, qW2, qb2, kW1, kb1, kW2, kb2):
    B, C, H, W = x.shape
    size = int(math.log2(C))
    s2 = size * size
    c4 = C // 4
    HW = H * W
    bh, bw = H // size, W // size
    assert H % size == 0 and W % size == 0, "even adaptive bins expected"
    assert bh & (bh - 1) == 0 and bw & (bw - 1) == 0, "pow2 bins expected"

    xf = x.astype(jnp.float32)

    ii, jj = np.divmod(np.arange(HW), W)
    bin_of = jnp.asarray((ii // bh) * size + (jj // bw), dtype=jnp.int32)
    corner = jnp.asarray((ii % bh == 0) & (jj % bw == 0))

    qw1m = qW1[:, :s2].T
    qw1a = qW1[:, s2:].T
    w1m = jnp.where(corner[:, None], qw1m[bin_of], 0.0).astype(jnp.bfloat16)
    w1a = (qw1a[bin_of] / float(bh * bw)).astype(jnp.bfloat16)

    qb1r = qb1.reshape(1, s2 // 2)
    qw2t = qW2.T
    qb2r = qb2.reshape(1, 1)
    kw1r = kW1.reshape(c4, C)
    kb1r = kb1.reshape(c4, 1)
    kw2r = kW2.reshape(C, c4)
    kb2r = kb2.reshape(C, 1)

    shifts = [1 << t for t in range(int(math.log2(bw)))]
    shifts += [W * (1 << t) for t in range(int(math.log2(bh)))]

    wts = [w1m, w1a, qb1r, qw2t, qb2r, kw1r, kb1r, kw2r, kb2r]

    out = pl.pallas_call(
        functools.partial(_fused_kernel, shifts=shifts, hw=HW, nb=B,
                          height=H, width=W),
        out_shape=jax.ShapeDtypeStruct((B, C, HW), jnp.float32),
        grid=(B,),
        in_specs=[pl.BlockSpec(memory_space=pl.ANY)] * (1 + len(wts)),
        out_specs=pl.BlockSpec(memory_space=pl.ANY),
        scratch_shapes=[
            pltpu.VMEM((2, C, HW), jnp.float32),
            pltpu.VMEM((2, C, HW), jnp.float32),
        ] + [pltpu.VMEM(w.shape, w.dtype) for w in wts] + [
            pltpu.SemaphoreType.DMA((2,)),
            pltpu.SemaphoreType.DMA((2,)),
            pltpu.SemaphoreType.DMA,
        ],
        compiler_params=pltpu.CompilerParams(
            dimension_semantics=("arbitrary",),
            vmem_limit_bytes=48 << 20),
        cost_estimate=pl.CostEstimate(
            flops=2 * B * C * HW * s2 + 4 * B * C * HW,
            transcendentals=B * C,
            bytes_accessed=2 * B * C * HW * 4),
    )(xf.reshape(B, C, HW), *wts)

    return out.reshape(B, C, H, W)


# speedup vs baseline: 1.0455x; 1.0455x over previous
import functools
import math

import numpy as np

import jax
import jax.numpy as jnp
from jax.experimental import pallas as pl
from jax.experimental.pallas import tpu as pltpu


def _fused_kernel(x_ref, w1m_ref, w1a_ref, qb1_ref, qw2_ref, qb2_ref,
                  kw1_ref, kb1_ref, kw2_ref, kb2_ref, o_ref,
                  *, shifts, hw, gsz, rows):
    c = x_ref.shape[1]
    q1s = []
    for g in range(gsz):
        parts = []
        for r0 in range(0, c, rows):
            xc = x_ref[g, r0:r0 + rows, :]
            v = xc
            for sh in shifts:
                v = jnp.maximum(v, pltpu.roll(v, hw - sh, 1))
            parts.append(
                jnp.dot(v.astype(jnp.bfloat16), w1m_ref[...],
                        preferred_element_type=jnp.float32)
                + jnp.dot(xc.astype(jnp.bfloat16), w1a_ref[...],
                          preferred_element_type=jnp.float32))
        q1s.append(jnp.maximum(
            jnp.concatenate(parts, axis=0) + qb1_ref[...], 0.0))

    gates = []
    for g in range(gsz):
        q2 = jnp.dot(q1s[g], qw2_ref[...],
                     preferred_element_type=jnp.float32) + qb2_ref[...]
        k1 = jnp.maximum(
            jnp.dot(kw1_ref[...], q2, preferred_element_type=jnp.float32)
            + kb1_ref[...], 0.0)
        k2 = jax.nn.sigmoid(
            jnp.dot(kw2_ref[...], k1, preferred_element_type=jnp.float32)
            + kb2_ref[...])
        gates.append(k2 + 1.0)

    for g in range(gsz):
        o_ref[g] = x_ref[g] * gates[g]


def kernel(x, qW1, qb1, qW2, qb2, kW1, kb1, kW2, kb2):
    B, C, H, W = x.shape
    size = int(math.log2(C))
    s2 = size * size
    c4 = C // 4
    HW = H * W
    bh, bw = H // size, W // size
    assert H % size == 0 and W % size == 0, "even adaptive bins expected"
    assert bh & (bh - 1) == 0 and bw & (bw - 1) == 0, "pow2 bins expected"

    xf = x.astype(jnp.float32)
    x3 = xf.reshape(B, C, HW)

    ii, jj = np.divmod(np.arange(HW), W)
    bin_of = jnp.asarray((ii // bh) * size + (jj // bw), dtype=jnp.int32)
    corner = jnp.asarray((ii % bh == 0) & (jj % bw == 0))

    qw1m = qW1[:, :s2].T
    qw1a = qW1[:, s2:].T
    w1m = jnp.where(corner[:, None], qw1m[bin_of], 0.0).astype(jnp.bfloat16)
    w1a = (qw1a[bin_of] / float(bh * bw)).astype(jnp.bfloat16)

    qb1r = qb1.reshape(1, s2 // 2)
    qw2t = qW2.T
    qb2r = qb2.reshape(1, 1)
    kw1r = kW1.reshape(c4, C)
    kb1r = kb1.reshape(c4, 1)
    kw2r = kW2.reshape(C, c4)
    kb2r = kb2.reshape(C, 1)

    shifts = [1 << t for t in range(int(math.log2(bw)))]
    shifts += [W * (1 << t) for t in range(int(math.log2(bh)))]

    def full(shape):
        return pl.BlockSpec(shape, lambda b, _n=len(shape): (0,) * _n)

    gsz = 4
    rows = 64

    out = pl.pallas_call(
        functools.partial(_fused_kernel, shifts=shifts, hw=HW,
                          gsz=gsz, rows=rows),
        out_shape=jax.ShapeDtypeStruct((B, C, HW), jnp.float32),
        grid=(B // gsz,),
        in_specs=[
            pl.BlockSpec((gsz, C, HW), lambda b: (b, 0, 0)),
            full((HW, s2 // 2)), full((HW, s2 // 2)),
            full((1, s2 // 2)),
            full((s2 // 2, 1)), full((1, 1)),
            full((c4, C)), full((c4, 1)),
            full((C, c4)), full((C, 1)),
        ],
        out_specs=pl.BlockSpec((gsz, C, HW), lambda b: (b, 0, 0)),
        compiler_params=pltpu.CompilerParams(
            dimension_semantics=("parallel",),
            vmem_limit_bytes=48 << 20),
        cost_estimate=pl.CostEstimate(
            flops=2 * B * C * HW * s2 + 4 * B * C * HW,
            transcendentals=B * C,
            bytes_accessed=2 * B * C * HW * 4),
    )(x3, w1m, w1a, qb1r, qw2t, qb2r, kw1r, kb1r, kw2r, kb2r)

    return out.reshape(B, C, H, W)


# speedup vs baseline: 1.1417x; 1.0920x over previous
import functools
import math

import numpy as np

import jax
import jax.numpy as jnp
from jax.experimental import pallas as pl
from jax.experimental.pallas import tpu as pltpu


def _fused_kernel(x_ref, w1m_ref, w1a_ref, qb1_ref, qw2_ref, qb2_ref,
                  kw1_ref, kb1_ref, kw2_ref, kb2_ref, o_ref,
                  *, shifts, hw, gsz, rows):
    c = x_ref.shape[1]
    q1s = []
    for g in range(gsz):
        parts = []
        for r0 in range(0, c, rows):
            xc = x_ref[g, r0:r0 + rows, :]
            v = xc
            for sh in shifts:
                v = jnp.maximum(v, jnp.concatenate(
                    [v[:, sh:], v[:, :sh]], axis=1))
            parts.append(
                jnp.dot(v.astype(jnp.bfloat16), w1m_ref[...],
                        preferred_element_type=jnp.float32)
                + jnp.dot(xc.astype(jnp.bfloat16), w1a_ref[...],
                          preferred_element_type=jnp.float32))
        q1s.append(jnp.maximum(
            jnp.concatenate(parts, axis=0) + qb1_ref[...], 0.0))

    gates = []
    for g in range(gsz):
        q2 = jnp.dot(q1s[g], qw2_ref[...],
                     preferred_element_type=jnp.float32) + qb2_ref[...]
        k1 = jnp.maximum(
            jnp.dot(kw1_ref[...], q2, preferred_element_type=jnp.float32)
            + kb1_ref[...], 0.0)
        k2 = jax.nn.sigmoid(
            jnp.dot(kw2_ref[...], k1, preferred_element_type=jnp.float32)
            + kb2_ref[...])
        gates.append(k2 + 1.0)

    for g in range(gsz):
        o_ref[g] = x_ref[g] * gates[g]


def kernel(x, qW1, qb1, qW2, qb2, kW1, kb1, kW2, kb2):
    B, C, H, W = x.shape
    size = int(math.log2(C))
    s2 = size * size
    c4 = C // 4
    HW = H * W
    bh, bw = H // size, W // size
    assert H % size == 0 and W % size == 0, "even adaptive bins expected"
    assert bh & (bh - 1) == 0 and bw & (bw - 1) == 0, "pow2 bins expected"

    xf = x.astype(jnp.float32)
    x3 = xf.reshape(B, C, HW)

    ii, jj = np.divmod(np.arange(HW), W)
    bin_of = jnp.asarray((ii // bh) * size + (jj // bw), dtype=jnp.int32)
    corner = jnp.asarray((ii % bh == 0) & (jj % bw == 0))

    qw1m = qW1[:, :s2].T
    qw1a = qW1[:, s2:].T
    w1m = jnp.where(corner[:, None], qw1m[bin_of], 0.0).astype(jnp.bfloat16)
    w1a = (qw1a[bin_of] / float(bh * bw)).astype(jnp.bfloat16)

    gsz = 8
    rows = 128

    qb1r = qb1.reshape(1, s2 // 2)
    qw2t = qW2.T
    qb2r = qb2.reshape(1, 1)
    kw1r = kW1.reshape(c4, C)
    kb1r = kb1.reshape(c4, 1)
    kw2r = kW2.reshape(C, c4)
    kb2r = kb2.reshape(C, 1)

    shifts = [1 << t for t in range(int(math.log2(bw)))]
    shifts += [W * (1 << t) for t in range(int(math.log2(bh)))]

    def full(shape):
        return pl.BlockSpec(shape, lambda b, _n=len(shape): (0,) * _n)

    out = pl.pallas_call(
        functools.partial(_fused_kernel, shifts=shifts, hw=HW,
                          gsz=gsz, rows=rows),
        out_shape=jax.ShapeDtypeStruct((B, C, HW), jnp.float32),
        grid=(B // gsz,),
        in_specs=[
            pl.BlockSpec((gsz, C, HW), lambda b: (b, 0, 0)),
            full((HW, s2 // 2)), full((HW, s2 // 2)),
            full((1, s2 // 2)),
            full((s2 // 2, 1)), full((1, 1)),
            full((c4, C)), full((c4, 1)),
            full((C, c4)), full((C, 1)),
        ],
        out_specs=pl.BlockSpec((gsz, C, HW), lambda b: (b, 0, 0)),
        compiler_params=pltpu.CompilerParams(
            dimension_semantics=("parallel",),
            vmem_limit_bytes=48 << 20),
        cost_estimate=pl.CostEstimate(
            flops=2 * B * C * HW * s2 + 4 * B * C * HW,
            transcendentals=B * C,
            bytes_accessed=2 * B * C * HW * 4),
    )(x3, w1m, w1a, qb1r, qw2t, qb2r, kw1r, kb1r, kw2r, kb2r)

    return out.reshape(B, C, H, W)
